# baseline (device time: 191376 ns/iter reference)
import jax
import jax.numpy as jnp
from jax import lax
from jax.experimental import pallas as pl
from jax.experimental.pallas import tpu as pltpu

N_DEV = 4
B_PER = 2
SQ = 512
SKV = 512
HG = 32
H_PER = 8
DH = 64
D_MODEL = 768
SCALE = 0.125


def kernel(x, Wq, K_ext, V_ext, Wo):
    def body(x_ref, wq_ref, k_hbm, v_hbm, wo_ref, out_ref,
             x_bf, wq_bf, wo_bf, wq_comm, wo_comm, ctx_ref, k_buf, v_buf,
             send_q, recv_q, send_o, recv_o, k_sems, v_sems):
        my_pos = lax.axis_index("i")
        bg0 = my_pos * B_PER

        kv_descs = []
        for c in range(N_DEV):
            origin = lax.rem(my_pos + c, N_DEV)
            hs = pl.ds(origin * H_PER, H_PER)
            dk = pltpu.make_async_copy(
                k_hbm.at[pl.ds(bg0, B_PER), :, hs], k_buf.at[c], k_sems.at[c])
            dv = pltpu.make_async_copy(
                v_hbm.at[pl.ds(bg0, B_PER), :, hs], v_buf.at[c], v_sems.at[c])
            dk.start()
            dv.start()
            kv_descs.append((dk, dv))

        x_bf[...] = (x_ref[...] * SCALE).astype(jnp.bfloat16)
        wq_bf[...] = wq_ref[...].astype(jnp.bfloat16)
        wo_bf[...] = wo_ref[...].astype(jnp.bfloat16)

        barrier_sem = pltpu.get_barrier_semaphore()
        for d in range(1, N_DEV):
            peer = lax.rem(my_pos + d, N_DEV)
            pl.semaphore_signal(
                barrier_sem, inc=1,
                device_id=(peer,), device_id_type=pl.DeviceIdType.MESH,
            )
        pl.semaphore_wait(barrier_sem, N_DEV - 1)

        HH = H_PER * DH // 2
        sends = []
        for half in range(2):
            cs = slice(half * HH, (half + 1) * HH)
            for d in range(1, N_DEV):
                target = lax.rem(my_pos + d, N_DEV)
                slot = N_DEV - 1 - d
                rq = pltpu.make_async_remote_copy(
                    src_ref=wq_bf.at[:, cs], dst_ref=wq_comm.at[slot, :, cs],
                    send_sem=send_q.at[d - 1, half], recv_sem=recv_q.at[slot, half],
                    device_id=(target,), device_id_type=pl.DeviceIdType.MESH,
                )
                ro = pltpu.make_async_remote_copy(
                    src_ref=wo_bf.at[cs], dst_ref=wo_comm.at[slot, cs],
                    send_sem=send_o.at[d - 1, half], recv_sem=recv_o.at[slot, half],
                    device_id=(target,), device_id_type=pl.DeviceIdType.MESH,
                )
                rq.start()
                ro.start()
                sends.append((rq, ro))

        qi = lax.broadcasted_iota(jnp.int32, (SQ, SKV), 0)
        ki = lax.broadcasted_iota(jnp.int32, (SQ, SKV), 1)
        mask = (jnp.abs(qi - ki) <= 128) | (ki < 32) | (qi < 32)
        bias = jnp.where(mask, 0.0, -1e9).astype(jnp.bfloat16)

        def compute_block(wq_c, wo_c, kslot, h_lo, n_h, first):
            for b in range(B_PER):
                q = lax.dot_general(
                    x_bf[b], wq_c, (((1,), (0,)), ((), ())),
                    preferred_element_type=jnp.float32,
                ).astype(jnp.bfloat16)
                for h in range(n_h):
                    qh = q[:, h * DH:(h + 1) * DH]
                    kh = k_buf[kslot, b, :, h_lo + h, :].astype(jnp.bfloat16)
                    s = lax.dot_general(
                        qh, kh, (((1,), (1,)), ((), ())),
                        preferred_element_type=jnp.float32,
                    ).astype(jnp.bfloat16)
                    w = jnp.exp(s + bias)
                    wsum = jnp.sum(w, axis=-1, keepdims=True,
                                   dtype=jnp.float32)
                    vh = v_buf[kslot, b, :, h_lo + h, :].astype(jnp.bfloat16)
                    ctx_h = lax.dot_general(
                        w, vh, (((1,), (0,)), ((), ())),
                        preferred_element_type=jnp.float32,
                    )
                    ctx_ref[:, h * DH:(h + 1) * DH] = (ctx_h / wsum).astype(jnp.bfloat16)
                partial = lax.dot_general(
                    ctx_ref[:, :n_h * DH], wo_c, (((1,), (0,)), ((), ())),
                    preferred_element_type=jnp.float32,
                )
                if first:
                    out_ref[b] = partial
                else:
                    out_ref[b] = out_ref[b] + partial

        kv_descs[0][0].wait()
        kv_descs[0][1].wait()
        compute_block(wq_bf[...], wo_bf[...], 0, 0, H_PER, first=True)

        for slot, half in ((2, 0), (0, 0), (2, 1), (0, 1), (1, 0), (1, 1)):
            cs = slice(half * HH, (half + 1) * HH)
            recv_desc_q = pltpu.make_async_remote_copy(
                src_ref=wq_bf.at[:, cs], dst_ref=wq_comm.at[slot, :, cs],
                send_sem=send_q.at[0, half], recv_sem=recv_q.at[slot, half],
                device_id=(my_pos,), device_id_type=pl.DeviceIdType.MESH,
            )
            recv_desc_o = pltpu.make_async_remote_copy(
                src_ref=wo_bf.at[cs], dst_ref=wo_comm.at[slot, cs],
                send_sem=send_o.at[0, half], recv_sem=recv_o.at[slot, half],
                device_id=(my_pos,), device_id_type=pl.DeviceIdType.MESH,
            )
            recv_desc_q.wait_recv()
            recv_desc_o.wait_recv()
            kslot = slot + 1
            if half == 0:
                kv_descs[kslot][0].wait()
                kv_descs[kslot][1].wait()
            compute_block(wq_comm[slot][:, cs], wo_comm[slot][cs],
                          kslot, half * (H_PER // 2), H_PER // 2, first=False)

        for rq, ro in sends:
            rq.wait_send()
            ro.wait_send()

    return pl.pallas_call(
        body,
        out_shape=jax.ShapeDtypeStruct((B_PER, SQ, D_MODEL), jnp.float32),
        in_specs=[
            pl.BlockSpec(memory_space=pltpu.VMEM),
            pl.BlockSpec(memory_space=pltpu.VMEM),
            pl.BlockSpec(memory_space=pl.ANY),
            pl.BlockSpec(memory_space=pl.ANY),
            pl.BlockSpec(memory_space=pltpu.VMEM),
        ],
        out_specs=pl.BlockSpec(memory_space=pltpu.VMEM),
        scratch_shapes=[
            pltpu.VMEM((B_PER, SQ, D_MODEL), jnp.bfloat16),
            pltpu.VMEM((D_MODEL, H_PER * DH), jnp.bfloat16),
            pltpu.VMEM((H_PER * DH, D_MODEL), jnp.bfloat16),
            pltpu.VMEM((N_DEV - 1, D_MODEL, H_PER * DH), jnp.bfloat16),
            pltpu.VMEM((N_DEV - 1, H_PER * DH, D_MODEL), jnp.bfloat16),
            pltpu.VMEM((SQ, H_PER * DH), jnp.bfloat16),
            pltpu.VMEM((N_DEV, B_PER, SKV, H_PER, DH), jnp.float32),
            pltpu.VMEM((N_DEV, B_PER, SKV, H_PER, DH), jnp.float32),
            pltpu.SemaphoreType.DMA((N_DEV - 1, 2)),
            pltpu.SemaphoreType.DMA((N_DEV - 1, 2)),
            pltpu.SemaphoreType.DMA((N_DEV - 1, 2)),
            pltpu.SemaphoreType.DMA((N_DEV - 1, 2)),
            pltpu.SemaphoreType.DMA((N_DEV,)),
            pltpu.SemaphoreType.DMA((N_DEV,)),
        ],
        compiler_params=pltpu.CompilerParams(
            collective_id=0,
            vmem_limit_bytes=112 * 1024 * 1024,
        ),
    )(x, Wq, K_ext, V_ext, Wo)


# device time: 87413 ns/iter; 2.1893x vs baseline; 2.1893x over previous
import jax
import jax.numpy as jnp
from jax import lax
from jax.experimental import pallas as pl
from jax.experimental.pallas import tpu as pltpu

N_DEV = 4
B_PER = 2
SQ = 512
SKV = 512
HG = 32
H_PER = 8
DH = 64
D_MODEL = 768
SCALE = 0.125


def kernel(x, Wq, K_ext, V_ext, Wo):
    my = lax.axis_index("i")

    k_my = lax.dynamic_slice(K_ext, (my * B_PER, 0, 0, 0), (B_PER, SKV, HG, DH))
    v_my = lax.dynamic_slice(V_ext, (my * B_PER, 0, 0, 0), (B_PER, SKV, HG, DH))
    k_r = k_my.astype(jnp.bfloat16).reshape(
        B_PER, SKV, N_DEV, H_PER, DH).transpose(2, 0, 3, 1, 4)
    v_r = v_my.astype(jnp.bfloat16).reshape(
        B_PER, SKV, N_DEV, H_PER, DH).transpose(2, 0, 3, 1, 4)
    x = (x * SCALE).astype(jnp.bfloat16)
    Wq = Wq.astype(jnp.bfloat16)
    Wo = Wo.astype(jnp.bfloat16)

    def body(x_ref, wq_ref, k_ref, v_ref, wo_ref, out_ref,
             wq_comm, wo_comm, ctx_ref, send_q, recv_q, send_o, recv_o):
        my_pos = lax.axis_index("i")

        barrier_sem = pltpu.get_barrier_semaphore()
        for d in range(1, N_DEV):
            peer = lax.rem(my_pos + d, N_DEV)
            pl.semaphore_signal(
                barrier_sem, inc=1,
                device_id=(peer,), device_id_type=pl.DeviceIdType.MESH,
            )
        pl.semaphore_wait(barrier_sem, N_DEV - 1)

        HH = H_PER * DH // 2
        sends = []
        for half in range(2):
            cs = slice(half * HH, (half + 1) * HH)
            for d in range(1, N_DEV):
                target = lax.rem(my_pos + d, N_DEV)
                slot = N_DEV - 1 - d
                rq = pltpu.make_async_remote_copy(
                    src_ref=wq_ref.at[:, cs], dst_ref=wq_comm.at[slot, :, cs],
                    send_sem=send_q.at[d - 1, half], recv_sem=recv_q.at[slot, half],
                    device_id=(target,), device_id_type=pl.DeviceIdType.MESH,
                )
                ro = pltpu.make_async_remote_copy(
                    src_ref=wo_ref.at[cs], dst_ref=wo_comm.at[slot, cs],
                    send_sem=send_o.at[d - 1, half], recv_sem=recv_o.at[slot, half],
                    device_id=(target,), device_id_type=pl.DeviceIdType.MESH,
                )
                rq.start()
                ro.start()
                sends.append((rq, ro))

        qi = lax.broadcasted_iota(jnp.int32, (SQ, SKV), 0)
        ki = lax.broadcasted_iota(jnp.int32, (SQ, SKV), 1)
        mask = (jnp.abs(qi - ki) <= 128) | (ki < 32) | (qi < 32)
        bias = jnp.where(mask, 0.0, -1e9).astype(jnp.bfloat16)

        def compute_block(wq_c, wo_c, origin, h_lo, n_h, first):
            for b in range(B_PER):
                q = lax.dot_general(
                    x_ref[b], wq_c, (((1,), (0,)), ((), ())),
                    preferred_element_type=jnp.float32,
                ).astype(jnp.bfloat16)
                for h in range(n_h):
                    qh = q[:, h * DH:(h + 1) * DH]
                    kh = k_ref[origin, b, h_lo + h]
                    s = lax.dot_general(
                        qh, kh, (((1,), (1,)), ((), ())),
                        preferred_element_type=jnp.float32,
                    ).astype(jnp.bfloat16)
                    w = jnp.exp(s + bias)
                    wsum = jnp.sum(w, axis=-1, keepdims=True,
                                   dtype=jnp.float32)
                    vh = v_ref[origin, b, h_lo + h]
                    ctx_h = lax.dot_general(
                        w, vh, (((1,), (0,)), ((), ())),
                        preferred_element_type=jnp.float32,
                    )
                    ctx_ref[:, h * DH:(h + 1) * DH] = (ctx_h / wsum).astype(jnp.bfloat16)
                partial = lax.dot_general(
                    ctx_ref[:, :n_h * DH], wo_c, (((1,), (0,)), ((), ())),
                    preferred_element_type=jnp.float32,
                )
                if first:
                    out_ref[b] = partial
                else:
                    out_ref[b] = out_ref[b] + partial

        compute_block(wq_ref[...], wo_ref[...], my_pos, 0, H_PER, first=True)

        for slot, half in ((2, 0), (0, 0), (2, 1), (0, 1), (1, 0), (1, 1)):
            cs = slice(half * HH, (half + 1) * HH)
            recv_desc_q = pltpu.make_async_remote_copy(
                src_ref=wq_ref.at[:, cs], dst_ref=wq_comm.at[slot, :, cs],
                send_sem=send_q.at[0, half], recv_sem=recv_q.at[slot, half],
                device_id=(my_pos,), device_id_type=pl.DeviceIdType.MESH,
            )
            recv_desc_o = pltpu.make_async_remote_copy(
                src_ref=wo_ref.at[cs], dst_ref=wo_comm.at[slot, cs],
                send_sem=send_o.at[0, half], recv_sem=recv_o.at[slot, half],
                device_id=(my_pos,), device_id_type=pl.DeviceIdType.MESH,
            )
            recv_desc_q.wait_recv()
            recv_desc_o.wait_recv()
            origin = lax.rem(my_pos + slot + 1, N_DEV)
            compute_block(wq_comm[slot][:, cs], wo_comm[slot][cs],
                          origin, half * (H_PER // 2), H_PER // 2, first=False)

        for rq, ro in sends:
            rq.wait_send()
            ro.wait_send()

    return pl.pallas_call(
        body,
        out_shape=jax.ShapeDtypeStruct((B_PER, SQ, D_MODEL), jnp.float32),
        in_specs=[
            pl.BlockSpec(memory_space=pltpu.VMEM),
            pl.BlockSpec(memory_space=pltpu.VMEM),
            pl.BlockSpec(memory_space=pltpu.VMEM),
            pl.BlockSpec(memory_space=pltpu.VMEM),
            pl.BlockSpec(memory_space=pltpu.VMEM),
        ],
        out_specs=pl.BlockSpec(memory_space=pltpu.VMEM),
        scratch_shapes=[
            pltpu.VMEM((N_DEV - 1, D_MODEL, H_PER * DH), jnp.bfloat16),
            pltpu.VMEM((N_DEV - 1, H_PER * DH, D_MODEL), jnp.bfloat16),
            pltpu.VMEM((SQ, H_PER * DH), jnp.bfloat16),
            pltpu.SemaphoreType.DMA((N_DEV - 1, 2)),
            pltpu.SemaphoreType.DMA((N_DEV - 1, 2)),
            pltpu.SemaphoreType.DMA((N_DEV - 1, 2)),
            pltpu.SemaphoreType.DMA((N_DEV - 1, 2)),
        ],
        compiler_params=pltpu.CompilerParams(
            collective_id=0,
            vmem_limit_bytes=100 * 1024 * 1024,
        ),
    )(x, Wq, k_r, v_r, Wo)
